# baseline (device time: 17814 ns/iter reference)
import jax
import jax.numpy as jnp
from jax import lax
from jax.experimental import pallas as pl
from jax.experimental.pallas import tpu as pltpu

T = 512
TG = T // 4
V_SHARD = 4096
D = 512


def kernel(ids, E):
    def body(ids_ref, e_ref, out_ref, zbuf, sbuf, rbuf,
             zsend, zrecv, bsend, brecv):
        my_x = lax.axis_index("x")
        my_y = lax.axis_index("y")
        my_z = lax.axis_index("z")
        my_g = my_x * 2 + my_y
        z_peer = (my_x, my_y, 1 - my_z)
        xy_peer = {
            1: (my_x, 1 - my_y, my_z),
            2: (1 - my_x, my_y, my_z),
            3: (1 - my_x, 1 - my_y, my_z),
        }

        barrier = pltpu.get_barrier_semaphore()
        for dev in [z_peer] + list(xy_peer.values()):
            pl.semaphore_signal(
                barrier, inc=1, device_id=dev,
                device_id_type=pl.DeviceIdType.MESH,
            )
        pl.semaphore_wait(barrier, 4)

        my_ids = ids_ref[pl.ds(my_g * TG, TG), :]
        local = my_ids - my_z * V_SHARD
        cols = lax.broadcasted_iota(jnp.int32, (TG, V_SHARD), 1)
        onehot = (cols == local).astype(jnp.bfloat16)
        partial = jnp.dot(
            onehot, e_ref[:, :].astype(jnp.bfloat16),
            preferred_element_type=jnp.float32,
        )

        zbuf[0, :, :] = partial.astype(jnp.bfloat16)
        zx = pltpu.make_async_remote_copy(
            src_ref=zbuf.at[0], dst_ref=zbuf.at[1],
            send_sem=zsend, recv_sem=zrecv,
            device_id=z_peer, device_id_type=pl.DeviceIdType.MESH,
        )
        zx.start()
        zx.wait()
        reduced = partial + zbuf[1, :, :].astype(jnp.float32)
        out_ref[pl.ds(my_g * TG, TG), :] = reduced
        sbuf[:, :] = reduced.astype(jnp.bfloat16)

        rdmas = []
        for k in (1, 2, 3):
            r = pltpu.make_async_remote_copy(
                src_ref=sbuf, dst_ref=rbuf.at[k],
                send_sem=bsend.at[k], recv_sem=brecv.at[k],
                device_id=xy_peer[k], device_id_type=pl.DeviceIdType.MESH,
            )
            r.start()
            rdmas.append(r)
        for k, r in zip((1, 2, 3), rdmas):
            r.wait()
            src_g = my_g ^ k
            out_ref[pl.ds(src_g * TG, TG), :] = (
                rbuf[k, :, :].astype(jnp.float32)
            )

    return pl.pallas_call(
        body,
        out_shape=jax.ShapeDtypeStruct((T, D), jnp.float32),
        in_specs=[
            pl.BlockSpec(memory_space=pltpu.VMEM),
            pl.BlockSpec(memory_space=pltpu.VMEM),
        ],
        out_specs=pl.BlockSpec(memory_space=pltpu.VMEM),
        scratch_shapes=[
            pltpu.VMEM((2, TG, D), jnp.bfloat16),
            pltpu.VMEM((TG, D), jnp.bfloat16),
            pltpu.VMEM((4, TG, D), jnp.bfloat16),
            pltpu.SemaphoreType.DMA,
            pltpu.SemaphoreType.DMA,
            pltpu.SemaphoreType.DMA((4,)),
            pltpu.SemaphoreType.DMA((4,)),
        ],
        compiler_params=pltpu.CompilerParams(collective_id=0),
    )(ids.reshape(T, 1), E)


# device time: 17168 ns/iter; 1.0376x vs baseline; 1.0376x over previous
import jax
import jax.numpy as jnp
from jax import lax
from jax.experimental import pallas as pl
from jax.experimental.pallas import tpu as pltpu

T = 512
TG = T // 4
C = TG // 2
V_SHARD = 4096
D = 512


def kernel(ids, E):
    def body(ids_ref, e_ref, out_ref, zbuf, sbuf, rbuf,
             zsend, zrecv, bsend, brecv):
        my_x = lax.axis_index("x")
        my_y = lax.axis_index("y")
        my_z = lax.axis_index("z")
        my_g = my_x * 2 + my_y
        z_peer = (my_x, my_y, 1 - my_z)
        xy_peer = {
            1: (my_x, 1 - my_y, my_z),
            2: (1 - my_x, my_y, my_z),
            3: (1 - my_x, 1 - my_y, my_z),
        }

        barrier = pltpu.get_barrier_semaphore()
        for dev in [z_peer] + list(xy_peer.values()):
            pl.semaphore_signal(
                barrier, inc=1, device_id=dev,
                device_id_type=pl.DeviceIdType.MESH,
            )
        pl.semaphore_wait(barrier, 4)

        e_bf16 = e_ref[:, :].astype(jnp.bfloat16)
        cols = lax.broadcasted_iota(jnp.int32, (C, V_SHARD), 1)

        zx = []
        partials = []
        for c in range(2):
            my_ids = ids_ref[pl.ds(my_g * TG + c * C, C), :]
            onehot = (cols == (my_ids - my_z * V_SHARD)).astype(jnp.bfloat16)
            partial = jnp.dot(onehot, e_bf16,
                              preferred_element_type=jnp.float32)
            partials.append(partial)
            zbuf[0, c, :, :] = partial.astype(jnp.bfloat16)
            r = pltpu.make_async_remote_copy(
                src_ref=zbuf.at[0, c], dst_ref=zbuf.at[1, c],
                send_sem=zsend.at[c], recv_sem=zrecv.at[c],
                device_id=z_peer, device_id_type=pl.DeviceIdType.MESH,
            )
            r.start()
            zx.append(r)

        bx = []
        for c in range(2):
            zx[c].wait()
            reduced = (
                partials[c] + zbuf[1, c, :, :].astype(jnp.float32)
            ).astype(jnp.bfloat16)
            out_ref[pl.ds(my_g * TG + c * C, C), :] = reduced
            sbuf[c, :, :] = reduced
            for k in (1, 2, 3):
                r = pltpu.make_async_remote_copy(
                    src_ref=sbuf.at[c], dst_ref=rbuf.at[c, k],
                    send_sem=bsend.at[c, k], recv_sem=brecv.at[c, k],
                    device_id=xy_peer[k],
                    device_id_type=pl.DeviceIdType.MESH,
                )
                r.start()
                bx.append((c, k, r))

        for c, k, r in bx:
            r.wait()
            src_g = my_g ^ k
            out_ref[pl.ds(src_g * TG + c * C, C), :] = rbuf[c, k, :, :]

    return pl.pallas_call(
        body,
        out_shape=jax.ShapeDtypeStruct((T, D), jnp.bfloat16),
        in_specs=[
            pl.BlockSpec(memory_space=pltpu.VMEM),
            pl.BlockSpec(memory_space=pltpu.VMEM),
        ],
        out_specs=pl.BlockSpec(memory_space=pltpu.VMEM),
        scratch_shapes=[
            pltpu.VMEM((2, 2, C, D), jnp.bfloat16),
            pltpu.VMEM((2, C, D), jnp.bfloat16),
            pltpu.VMEM((2, 4, C, D), jnp.bfloat16),
            pltpu.SemaphoreType.DMA((2,)),
            pltpu.SemaphoreType.DMA((2,)),
            pltpu.SemaphoreType.DMA((2, 4)),
            pltpu.SemaphoreType.DMA((2, 4)),
        ],
        compiler_params=pltpu.CompilerParams(collective_id=0),
    )(ids.reshape(T, 1), E)
